# Initial kernel scaffold; baseline (speedup 1.0000x reference)
import jax
import jax.numpy as jnp
from jax.experimental import pallas as pl


def kernel(x_feat, pos, adj, down_w_0, down_w_1, down_w_2, down_b_0, down_b_1, down_b_2, pool_p_0, pool_p_1, up_w_0, up_w_1, up_b_0, up_b_1):
    raise NotImplementedError("write your pallas kernel here")


# speedup vs baseline: 1.9805x; 1.9805x over previous
import math
from functools import partial

import jax
import jax.numpy as jnp
from jax.experimental import pallas as pl
from jax.experimental.pallas import tpu as pltpu


def _round_up(v, m):
    return ((v + m - 1) // m) * m


def _gcn_body(*refs, relu, has_score, softmax_classes, tile):
    if has_score:
        x_ref, adj_ref, w_ref, b_ref, p_ref, o_ref, s_ref = refs
    else:
        x_ref, adj_ref, w_ref, b_ref, o_ref = refs
        p_ref = s_ref = None

    i = pl.program_id(0)

    adj = adj_ref[...]
    deg = jnp.sum(adj, axis=1, keepdims=True) + 2.0
    dinv = jax.lax.rsqrt(deg)

    w = w_ref[...]
    xw = jnp.dot(x_ref[...], w,
                 preferred_element_type=jnp.float32)
    xw_s = xw * dinv

    adj_t = adj_ref[pl.ds(i * tile, tile), :]
    x_t = x_ref[pl.ds(i * tile, tile), :]
    xw_t = jnp.dot(x_t, w, preferred_element_type=jnp.float32)
    deg_t = jnp.sum(adj_t, axis=1, keepdims=True) + 2.0
    dinv_t = jax.lax.rsqrt(deg_t)

    prop = jnp.dot(adj_t, xw_s, preferred_element_type=jnp.float32)
    out = (prop + 2.0 * xw_t * dinv_t) * dinv_t + b_ref[...]
    if relu:
        out = jnp.maximum(out, 0.0)

    if softmax_classes is not None:
        col = jax.lax.broadcasted_iota(jnp.int32, out.shape, 1)
        logits = jnp.where(col < softmax_classes, out, -jnp.inf)
        m = jnp.max(logits, axis=-1, keepdims=True)
        s = logits - m
        lse = jnp.log(jnp.sum(jnp.exp(s), axis=-1, keepdims=True))
        out = s - lse

    o_ref[...] = out.astype(o_ref.dtype)

    if has_score:
        raw = jnp.sum(out * p_ref[...], axis=-1, keepdims=True)
        s_ref[...] = jnp.tanh(raw)


def _bbt_body(brow_ref, ball_ref, o_ref, *, tile):
    i = pl.program_id(0)
    acc = jax.lax.dot_general(
        brow_ref[...], ball_ref[...],
        (((1,), (1,)), ((), ())),
        preferred_element_type=jnp.float32)
    r = jax.lax.broadcasted_iota(jnp.int32, acc.shape, 0) + i * tile
    c = jax.lax.broadcasted_iota(jnp.int32, acc.shape, 1)
    o_ref[...] = jnp.where(r == c, 0.0, acc)


def _gcn_conv(x, adj, w, b, *, relu, p_unit=None, softmax_classes=None,
              n_tiles=2):
    M = adj.shape[0]
    Cin = x.shape[1]
    Cout = w.shape[1]
    tile = M // n_tiles
    b2 = b.reshape(1, Cout).astype(jnp.float32)
    has_score = p_unit is not None

    inputs = [x.astype(jnp.float32), adj, w.astype(jnp.float32), b2]
    in_specs = [
        pl.BlockSpec((M, Cin), lambda i: (0, 0)),
        pl.BlockSpec((M, M), lambda i: (0, 0)),
        pl.BlockSpec((Cin, Cout), lambda i: (0, 0)),
        pl.BlockSpec((1, Cout), lambda i: (0, 0)),
    ]
    if has_score:
        inputs.append(p_unit.reshape(1, Cout).astype(jnp.float32))
        in_specs.append(pl.BlockSpec((1, Cout), lambda i: (0, 0)))
        out_shape = (jax.ShapeDtypeStruct((M, Cout), jnp.float32),
                     jax.ShapeDtypeStruct((M, 1), jnp.float32))
        out_specs = (pl.BlockSpec((tile, Cout), lambda i: (i, 0)),
                     pl.BlockSpec((tile, 1), lambda i: (i, 0)))
    else:
        out_shape = jax.ShapeDtypeStruct((M, Cout), jnp.float32)
        out_specs = pl.BlockSpec((tile, Cout), lambda i: (i, 0))

    return pl.pallas_call(
        partial(_gcn_body, relu=relu, has_score=has_score,
                softmax_classes=softmax_classes, tile=tile),
        out_shape=out_shape,
        grid=(n_tiles,),
        in_specs=in_specs,
        out_specs=out_specs,
        compiler_params=pltpu.CompilerParams(
            dimension_semantics=("parallel",)),
    )(*inputs)


def _bbt(bmat, n_tiles=8):
    Mp, K = bmat.shape
    tile = Mp // n_tiles
    return pl.pallas_call(
        partial(_bbt_body, tile=tile),
        out_shape=jax.ShapeDtypeStruct((Mp, Mp), jnp.float32),
        grid=(n_tiles,),
        in_specs=[pl.BlockSpec((tile, K), lambda i: (i, 0)),
                  pl.BlockSpec((Mp, K), lambda i: (0, 0))],
        out_specs=pl.BlockSpec((tile, Mp), lambda i: (i, 0)),
        compiler_params=pltpu.CompilerParams(
            dimension_semantics=("parallel",)),
    )(bmat, bmat)


def kernel(x_feat, pos, adj,
           down_w_0, down_w_1, down_w_2,
           down_b_0, down_b_1, down_b_2,
           pool_p_0, pool_p_1,
           up_w_0, up_w_1, up_b_0, up_b_1):
    N = adj.shape[0]
    num_classes = up_w_1.shape[1]
    adj = adj.astype(jnp.float32)

    p0_unit = pool_p_0 / jnp.linalg.norm(pool_p_0)
    p1_unit = pool_p_1 / jnp.linalg.norm(pool_p_1)

    x0 = jnp.concatenate([x_feat, pos], axis=-1).astype(jnp.float32)
    x0out, score0 = _gcn_conv(x0, adj, down_w_0, down_b_0,
                              relu=True, p_unit=p0_unit)

    n0 = N
    k1 = int(math.ceil(0.8 * n0))
    kpad1 = min(_round_up(k1, 8), N)
    Mp1 = _round_up(kpad1, 128)
    perm1 = jnp.argsort(-score0[:, 0])[:kpad1]
    perm1 = jnp.concatenate(
        [perm1, jnp.zeros((Mp1 - kpad1,), perm1.dtype)])
    gate1 = (jnp.arange(Mp1) < k1).astype(jnp.float32)

    sc1 = score0[perm1, 0] * gate1
    x1 = x0out[perm1] * sc1[:, None]
    eye_rows = (perm1[:, None] == jnp.arange(N)[None, :]).astype(jnp.float32)
    b1 = ((adj[perm1] + eye_rows) * gate1[:, None]).astype(jnp.bfloat16)

    adj1 = _bbt(b1)

    x1out, score1 = _gcn_conv(x1, adj1, down_w_1, down_b_1,
                              relu=True, p_unit=p1_unit)

    n1 = k1
    k2 = int(math.ceil(0.8 * n1))
    kpad2 = min(_round_up(k2, 8), kpad1)
    Mp2 = _round_up(kpad2, 128)
    valid = jnp.arange(Mp1) < n1
    masked = jnp.where(valid, score1[:, 0], -jnp.inf)
    perm2 = jnp.argsort(-masked)[:kpad2]
    perm2 = jnp.concatenate(
        [perm2, jnp.zeros((Mp2 - kpad2,), perm2.dtype)])
    gate2 = (jnp.arange(Mp2) < k2).astype(jnp.float32)

    sc2 = score1[perm2, 0] * gate2
    x2 = x1out[perm2] * sc2[:, None]
    eye_rows2 = (perm2[:, None] == jnp.arange(Mp1)[None, :]).astype(jnp.float32)
    b2 = ((adj1[perm2] + eye_rows2) * gate2[:, None]).astype(jnp.bfloat16)

    adj2 = _bbt(b2)

    x2out = _gcn_conv(x2, adj2, down_w_2, down_b_2, relu=True)

    c2 = x2out.shape[1]
    up1 = jnp.zeros((Mp1, c2), jnp.float32).at[perm2[:k2]].set(x2out[:k2])
    xc1 = jnp.concatenate([x1out, up1], axis=-1)
    xu1 = _gcn_conv(xc1, adj1, up_w_0, up_b_0, relu=True)

    c1 = xu1.shape[1]
    up0 = jnp.zeros((N, c1), jnp.float32).at[perm1[:k1]].set(xu1[:k1])
    xc0 = jnp.concatenate([x0out, up0], axis=-1)
    w_pad = jnp.zeros((up_w_1.shape[0], 128),
                      jnp.float32).at[:, :num_classes].set(up_w_1)
    b_pad = jnp.zeros((128,), jnp.float32).at[:num_classes].set(up_b_1)
    out = _gcn_conv(xc0, adj, w_pad, b_pad, relu=False,
                    softmax_classes=num_classes)
    return out[:, :num_classes]


# speedup vs baseline: 1.9927x; 1.0062x over previous
import math
from functools import partial

import jax
import jax.numpy as jnp
from jax.experimental import pallas as pl
from jax.experimental.pallas import tpu as pltpu


def _round_up(v, m):
    return ((v + m - 1) // m) * m


def _prep_body(adj_ref, a0_ref, asl_ref, dinv_ref, *, tile):
    i = pl.program_id(0)
    a = adj_ref[...]
    deg = jnp.sum(a, axis=1, keepdims=True) + 2.0
    dinv_ref[...] = jax.lax.rsqrt(deg)
    ab = a.astype(jnp.bfloat16)
    a0_ref[...] = ab
    r = jax.lax.broadcasted_iota(jnp.int32, a.shape, 0) + i * tile
    c = jax.lax.broadcasted_iota(jnp.int32, a.shape, 1)
    asl_ref[...] = jnp.where(r == c, 1.0, ab)


def _gcn_body(*refs, relu, has_score, softmax_classes, tile):
    if has_score:
        x_ref, adj_ref, dinv_ref, w_ref, b_ref, p_ref, o_ref, s_ref = refs
    else:
        x_ref, adj_ref, dinv_ref, w_ref, b_ref, o_ref = refs
        p_ref = s_ref = None

    i = pl.program_id(0)
    dinv = dinv_ref[...]
    w = w_ref[...]
    xw = jnp.dot(x_ref[...].astype(jnp.bfloat16), w,
                 preferred_element_type=jnp.float32)
    xw_s = (xw * dinv).astype(jnp.bfloat16)

    adj_t = adj_ref[...]
    x_t = x_ref[pl.ds(i * tile, tile), :]
    xw_t = jnp.dot(x_t.astype(jnp.bfloat16), w,
                   preferred_element_type=jnp.float32)
    dinv_t = dinv_ref[pl.ds(i * tile, tile), :]

    prop = jnp.dot(adj_t, xw_s, preferred_element_type=jnp.float32)
    out = (prop + 2.0 * xw_t * dinv_t) * dinv_t + b_ref[...]
    if relu:
        out = jnp.maximum(out, 0.0)

    if softmax_classes is not None:
        col = jax.lax.broadcasted_iota(jnp.int32, out.shape, 1)
        logits = jnp.where(col < softmax_classes, out, -jnp.inf)
        m = jnp.max(logits, axis=-1, keepdims=True)
        s = logits - m
        lse = jnp.log(jnp.sum(jnp.exp(s), axis=-1, keepdims=True))
        out = s - lse

    o_ref[...] = out.astype(o_ref.dtype)

    if has_score:
        raw = jnp.sum(out * p_ref[...], axis=-1, keepdims=True)
        s_ref[...] = jnp.tanh(raw)


def _bbt_body(brow_ref, ball_ref, gcol_ref, grow_ref,
              a0_ref, asl_ref, dinv_ref, *, tile):
    i = pl.program_id(0)
    acc = jax.lax.dot_general(
        brow_ref[...], ball_ref[...],
        (((1,), (1,)), ((), ())),
        preferred_element_type=jnp.float32)
    r = jax.lax.broadcasted_iota(jnp.int32, acc.shape, 0) + i * tile
    c = jax.lax.broadcasted_iota(jnp.int32, acc.shape, 1)
    diag = r == c
    out = jnp.where(diag, 0.0, acc) * (gcol_ref[...] * grow_ref[...])
    ob = out.astype(jnp.bfloat16)
    a0_ref[...] = ob
    asl_ref[...] = jnp.where(diag, 1.0, ob)
    deg = jnp.sum(out, axis=1, keepdims=True) + 2.0
    dinv_ref[...] = jax.lax.rsqrt(deg)


_PARALLEL = pltpu.CompilerParams(dimension_semantics=("parallel",))


def _prep(adj, n_tiles=4):
    N = adj.shape[0]
    tile = N // n_tiles
    return pl.pallas_call(
        partial(_prep_body, tile=tile),
        out_shape=(jax.ShapeDtypeStruct((N, N), jnp.bfloat16),
                   jax.ShapeDtypeStruct((N, N), jnp.bfloat16),
                   jax.ShapeDtypeStruct((N, 1), jnp.float32)),
        grid=(n_tiles,),
        in_specs=[pl.BlockSpec((tile, N), lambda i: (i, 0))],
        out_specs=(pl.BlockSpec((tile, N), lambda i: (i, 0)),
                   pl.BlockSpec((tile, N), lambda i: (i, 0)),
                   pl.BlockSpec((tile, 1), lambda i: (i, 0))),
        compiler_params=_PARALLEL,
    )(adj)


def _gcn_conv(x, adj_bf, dinv, w, b, *, relu, p_unit=None,
              softmax_classes=None, n_tiles=4):
    M = adj_bf.shape[0]
    Cin = x.shape[1]
    Cout = w.shape[1]
    tile = M // n_tiles
    b2 = b.reshape(1, Cout).astype(jnp.float32)
    has_score = p_unit is not None

    inputs = [x.astype(jnp.float32), adj_bf, dinv,
              w.astype(jnp.bfloat16), b2]
    in_specs = [
        pl.BlockSpec((M, Cin), lambda i: (0, 0)),
        pl.BlockSpec((tile, M), lambda i: (i, 0)),
        pl.BlockSpec((M, 1), lambda i: (0, 0)),
        pl.BlockSpec((Cin, Cout), lambda i: (0, 0)),
        pl.BlockSpec((1, Cout), lambda i: (0, 0)),
    ]
    if has_score:
        inputs.append(p_unit.reshape(1, Cout).astype(jnp.float32))
        in_specs.append(pl.BlockSpec((1, Cout), lambda i: (0, 0)))
        out_shape = (jax.ShapeDtypeStruct((M, Cout), jnp.float32),
                     jax.ShapeDtypeStruct((M, 1), jnp.float32))
        out_specs = (pl.BlockSpec((tile, Cout), lambda i: (i, 0)),
                     pl.BlockSpec((tile, 1), lambda i: (i, 0)))
    else:
        out_shape = jax.ShapeDtypeStruct((M, Cout), jnp.float32)
        out_specs = pl.BlockSpec((tile, Cout), lambda i: (i, 0))

    return pl.pallas_call(
        partial(_gcn_body, relu=relu, has_score=has_score,
                softmax_classes=softmax_classes, tile=tile),
        out_shape=out_shape,
        grid=(n_tiles,),
        in_specs=in_specs,
        out_specs=out_specs,
        compiler_params=_PARALLEL,
    )(*inputs)


def _bbt(bmat, gate, n_tiles=8):
    Mp, K = bmat.shape
    tile = Mp // n_tiles
    gcol = gate.reshape(Mp, 1)
    grow = gate.reshape(1, Mp)
    return pl.pallas_call(
        partial(_bbt_body, tile=tile),
        out_shape=(jax.ShapeDtypeStruct((Mp, Mp), jnp.bfloat16),
                   jax.ShapeDtypeStruct((Mp, Mp), jnp.bfloat16),
                   jax.ShapeDtypeStruct((Mp, 1), jnp.float32)),
        grid=(n_tiles,),
        in_specs=[pl.BlockSpec((tile, K), lambda i: (i, 0)),
                  pl.BlockSpec((Mp, K), lambda i: (0, 0)),
                  pl.BlockSpec((tile, 1), lambda i: (i, 0)),
                  pl.BlockSpec((1, Mp), lambda i: (0, 0))],
        out_specs=(pl.BlockSpec((tile, Mp), lambda i: (i, 0)),
                   pl.BlockSpec((tile, Mp), lambda i: (i, 0)),
                   pl.BlockSpec((tile, 1), lambda i: (i, 0))),
        compiler_params=_PARALLEL,
    )(bmat, bmat, gcol, grow)


def kernel(x_feat, pos, adj,
           down_w_0, down_w_1, down_w_2,
           down_b_0, down_b_1, down_b_2,
           pool_p_0, pool_p_1,
           up_w_0, up_w_1, up_b_0, up_b_1):
    N = adj.shape[0]
    num_classes = up_w_1.shape[1]

    p0_unit = pool_p_0 / jnp.linalg.norm(pool_p_0)
    p1_unit = pool_p_1 / jnp.linalg.norm(pool_p_1)

    a0bf, asl0, dinv0 = _prep(adj.astype(jnp.float32))

    x0 = jnp.concatenate([x_feat, pos], axis=-1).astype(jnp.float32)
    x0out, score0 = _gcn_conv(x0, a0bf, dinv0, down_w_0, down_b_0,
                              relu=True, p_unit=p0_unit)

    n0 = N
    k1 = int(math.ceil(0.8 * n0))
    kpad1 = min(_round_up(k1, 8), N)
    Mp1 = _round_up(kpad1, 128)
    perm1 = jnp.argsort(-score0[:, 0])[:kpad1]
    perm1 = jnp.concatenate(
        [perm1, jnp.zeros((Mp1 - kpad1,), perm1.dtype)])
    gate1 = (jnp.arange(Mp1) < k1).astype(jnp.float32)

    sc1 = score0[perm1, 0] * gate1
    x1 = x0out[perm1] * sc1[:, None]
    b1 = asl0[perm1]

    adj1, asl1, dinv1 = _bbt(b1, gate1)

    x1out, score1 = _gcn_conv(x1, adj1, dinv1, down_w_1, down_b_1,
                              relu=True, p_unit=p1_unit)

    n1 = k1
    k2 = int(math.ceil(0.8 * n1))
    kpad2 = min(_round_up(k2, 8), kpad1)
    Mp2 = _round_up(kpad2, 128)
    valid = jnp.arange(Mp1) < n1
    masked = jnp.where(valid, score1[:, 0], -jnp.inf)
    perm2 = jnp.argsort(-masked)[:kpad2]
    perm2 = jnp.concatenate(
        [perm2, jnp.zeros((Mp2 - kpad2,), perm2.dtype)])
    gate2 = (jnp.arange(Mp2) < k2).astype(jnp.float32)

    sc2 = score1[perm2, 0] * gate2
    x2 = x1out[perm2] * sc2[:, None]
    b2 = asl1[perm2]

    adj2, _, dinv2 = _bbt(b2, gate2)

    x2out = _gcn_conv(x2, adj2, dinv2, down_w_2, down_b_2, relu=True)

    c2 = x2out.shape[1]
    up1 = jnp.zeros((Mp1, c2), jnp.float32).at[perm2[:k2]].set(x2out[:k2])
    xc1 = jnp.concatenate([x1out, up1], axis=-1)
    xu1 = _gcn_conv(xc1, adj1, dinv1, up_w_0, up_b_0, relu=True)

    c1 = xu1.shape[1]
    up0 = jnp.zeros((N, c1), jnp.float32).at[perm1[:k1]].set(xu1[:k1])
    xc0 = jnp.concatenate([x0out, up0], axis=-1)
    w_pad = jnp.zeros((up_w_1.shape[0], 128),
                      jnp.float32).at[:, :num_classes].set(up_w_1)
    b_pad = jnp.zeros((128,), jnp.float32).at[:num_classes].set(up_b_1)
    out = _gcn_conv(xc0, a0bf, dinv0, w_pad, b_pad, relu=False,
                    softmax_classes=num_classes)
    return out[:, :num_classes]
